# baseline (speedup 1.0000x reference)
import functools

import jax
import jax.numpy as jnp
from jax import lax
from jax.experimental import pallas as pl
from jax.experimental.pallas import tpu as pltpu
from jax.experimental.pallas import tpu_sc as plsc

N = 10000
F = 128
H = 8
HH = 4
FH = HH * 16
C = 16
NEG = 0.2
NC = 2
NS = 16
NP = N + 112
K = 128
ZR = NP // NS

f32 = jnp.float32

_mesh = plsc.VectorSubcoreMesh(
    core_axis_name="c", subcore_axis_name="s", num_cores=NC, num_subcores=NS)


def _edge_pass(xl2, xr2, src, dst, att, nchunk):
  epw = nchunk * K

  @functools.partial(
      pl.kernel,
      out_type=(
          jax.ShapeDtypeStruct((NC, NP, FH), f32),
          jax.ShapeDtypeStruct((NC, NP, C), f32),
      ),
      mesh=_mesh,
      compiler_params=pltpu.CompilerParams(use_tc_tiling_on_sc=False),
      scratch_types=[
          pltpu.VMEM((K,), jnp.int32),
          pltpu.VMEM((K,), jnp.int32),
          pltpu.VMEM((K,), jnp.int32),
          pltpu.VMEM((K, FH), f32),
          pltpu.VMEM((K, FH), f32),
          pltpu.VMEM((K, FH), f32),
          pltpu.VMEM((K, C), f32),
          pltpu.VMEM((H, C), f32),
          pltpu.VMEM_SHARED((NP, FH), f32),
          pltpu.VMEM_SHARED((NP, C), f32),
          pltpu.SemaphoreType.DMA,
          pltpu.SemaphoreType.DMA,
      ],
  )
  def body(xl_h, xr_h, src_h, dst_h, att_h, wacc_h, den_h,
           srcv, dstv, dstv2, xlv, xrv, wv, eev, attv,
           wacc_sh, den_sh, sem1, sem2):
    cid = lax.axis_index("c")
    sid = lax.axis_index("s")

    zvec = jnp.zeros((C,), f32)

    def zrow(e, carry):
      for j in range(FH // C):
        wv[e, pl.ds(j * C, C)] = zvec
      eev[e, :] = zvec
      return carry

    lax.fori_loop(0, K, zrow, None)

    zb = sid * ZR
    nfull = ZR // K
    rem = ZR - nfull * K
    for i in range(nfull):
      pltpu.sync_copy(wv, wacc_sh.at[pl.ds(zb + i * K, K)])
      pltpu.sync_copy(eev, den_sh.at[pl.ds(zb + i * K, K)])
    if rem:
      pltpu.sync_copy(wv.at[pl.ds(0, rem)], wacc_sh.at[pl.ds(zb + nfull * K, rem)])
      pltpu.sync_copy(eev.at[pl.ds(0, rem)], den_sh.at[pl.ds(zb + nfull * K, rem)])
    pltpu.sync_copy(att_h, attv)
    plsc.subcore_barrier()

    lane = lax.iota(jnp.int32, C)
    perm8, perm4, perm2, perm1 = (lane ^ 8, lane ^ 4, lane ^ 2, lane ^ 1)
    mask8 = lane < 8
    mask4 = (lane & 4) == 0
    bidx = [jnp.full((C,), (h % 2) * 8 + (h // 2) * 4, jnp.int32)
            for h in range(HH)]

    def bfly(v, p):
      return v + jnp.take_along_axis(v, p, axis=0)

    shift = jnp.full((C,), cid * NP, jnp.int32)
    hbase = cid * HH
    att_regs = [attv[hbase + h, :] for h in range(HH)]

    def chunk_body(k, carry):
      eb = sid * epw + k * K
      pltpu.sync_copy(src_h.at[pl.ds(eb, K)], srcv)
      pltpu.sync_copy(dst_h.at[pl.ds(eb, K)], dstv)

      def shift_body(j, carry2):
        srcv[pl.ds(j * C, C)] = srcv[pl.ds(j * C, C)] + shift
        dstv2[pl.ds(j * C, C)] = dstv[pl.ds(j * C, C)] + shift
        return carry2

      lax.fori_loop(0, K // C, shift_body, None)

      cp1 = pltpu.async_copy(xl_h.at[srcv], xlv, sem1)
      cp2 = pltpu.async_copy(xr_h.at[dstv2], xrv, sem2)
      cp1.wait()
      cp2.wait()

      @plsc.parallel_loop(0, K, unroll=8)
      def edge_body(e):
        a = [xlv[e, pl.ds(h * C, C)] for h in range(HH)]
        p = []
        for h in range(HH):
          b = xrv[e, pl.ds(h * C, C)]
          s = a[h] + b
          m = jnp.maximum(s, NEG * s)
          p.append(m * att_regs[h])
        v = [bfly(x, perm8) for x in p]
        m01 = jnp.where(mask8, v[0], v[1])
        m23 = jnp.where(mask8, v[2], v[3])
        q = jnp.where(mask4, bfly(m01, perm4), bfly(m23, perm4))
        ev4 = jnp.exp(bfly(bfly(q, perm2), perm1))
        eev[e, :] = ev4
        for h in range(HH):
          wv[e, pl.ds(h * C, C)] = a[h] * jnp.take_along_axis(ev4, bidx[h], axis=0)

      pltpu.sync_copy(wv, wacc_sh.at[dstv], add=True)
      pltpu.sync_copy(eev, den_sh.at[dstv], add=True)
      return carry

    lax.fori_loop(0, nchunk, chunk_body, None)
    plsc.subcore_barrier()
    pltpu.sync_copy(wacc_sh.at[pl.ds(zb, ZR)], wacc_h.at[cid, pl.ds(zb, ZR)])
    pltpu.sync_copy(den_sh.at[pl.ds(zb, ZR)], den_h.at[cid, pl.ds(zb, ZR)])

  return body(xl2, xr2, src, dst, att)


def _mm2(xp, Wl, Wr):
  def bodyfn(x_ref, wl_ref, wr_ref, xl_ref, xr_ref):
    xv = x_ref[...]
    xl = jnp.dot(xv, wl_ref[...], preferred_element_type=f32)
    xr = jnp.dot(xv, wr_ref[...], preferred_element_type=f32)
    xl_ref[...] = jnp.concatenate([xl[:, :FH], xl[:, FH:]], axis=0)
    xr_ref[...] = jnp.concatenate([xr[:, :FH], xr[:, FH:]], axis=0)

  return pl.pallas_call(
      bodyfn,
      out_shape=(jax.ShapeDtypeStruct((NC * NP, FH), f32),
                 jax.ShapeDtypeStruct((NC * NP, FH), f32)),
  )(xp, Wl, Wr)


def _combine(wa_ref, dn_ref, s_ref, b_ref):
  wa = jnp.concatenate([wa_ref[0], wa_ref[1]], axis=1)
  rows = lax.broadcasted_iota(jnp.int32, (C, HH), 0)
  cols = lax.broadcasted_iota(jnp.int32, (C, HH), 1)
  sel = (rows == (cols % 2) * 8 + (cols // 2) * 4).astype(f32)
  d = jnp.concatenate(
      [jnp.dot(dn_ref[0], sel, preferred_element_type=f32),
       jnp.dot(dn_ref[1], sel, preferred_element_type=f32)], axis=1)
  r = 1.0 / (d + 1e-16)
  rexp = jnp.dot(r, s_ref[...], preferred_element_type=f32)
  return wa * rexp + b_ref[...]


def _combine_mm(wacc, den, S, b, Wl, Wr):
  def bodyfn(wa_ref, dn_ref, s_ref, b_ref, wl_ref, wr_ref, xl_ref, xr_ref):
    h = _combine(wa_ref, dn_ref, s_ref, b_ref)
    h = jnp.where(h > 0, h, jnp.exp(h) - 1.0)
    xl = jnp.dot(h, wl_ref[...], preferred_element_type=f32)
    xr = jnp.dot(h, wr_ref[...], preferred_element_type=f32)
    xl_ref[...] = jnp.concatenate([xl[:, :FH], xl[:, FH:]], axis=0)
    xr_ref[...] = jnp.concatenate([xr[:, :FH], xr[:, FH:]], axis=0)

  return pl.pallas_call(
      bodyfn,
      out_shape=(jax.ShapeDtypeStruct((NC * NP, FH), f32),
                 jax.ShapeDtypeStruct((NC * NP, FH), f32)),
  )(wacc, den, S, b, Wl, Wr)


def _finalize(wacc, den, S, b):
  def bodyfn(wa_ref, dn_ref, s_ref, b_ref, o_ref):
    h = _combine(wa_ref, dn_ref, s_ref, b_ref)
    m = jnp.max(h, axis=1, keepdims=True)
    z = h - m
    o_ref[...] = z - jnp.log(jnp.sum(jnp.exp(z), axis=1, keepdims=True))

  return pl.pallas_call(
      bodyfn,
      out_shape=jax.ShapeDtypeStruct((NP, F), f32),
  )(wacc, den, S, b)


def kernel(x, edge_index, W1l, W1r, att1, b1, W2l, W2r, att2, b2,
           W3l, W3r, att3, b3):
  eg = edge_index.shape[1]
  etot = eg + N
  nchunk = -(-etot // (NS * K))
  ep = NS * nchunk * K

  loop = jnp.arange(N, dtype=jnp.int32)
  padi = jnp.full((ep - etot,), N, jnp.int32)
  src = jnp.concatenate([edge_index[0].astype(jnp.int32), loop, padi])
  dst = jnp.concatenate([edge_index[1].astype(jnp.int32), loop, padi])

  xp = jnp.pad(x.astype(f32), ((0, NP - N), (0, 0)))

  S = (jnp.arange(F)[None, :] // C == jnp.arange(H)[:, None]).astype(f32)

  xl, xr = _mm2(xp, W1l, W1r)
  wacc, den = _edge_pass(xl, xr, src, dst, att1, nchunk)
  xl, xr = _combine_mm(wacc, den, S, b1.reshape(1, F), W2l, W2r)
  wacc, den = _edge_pass(xl, xr, src, dst, att2, nchunk)
  xl, xr = _combine_mm(wacc, den, S, b2.reshape(1, F), W3l, W3r)
  wacc, den = _edge_pass(xl, xr, src, dst, att3, nchunk)
  out = _finalize(wacc, den, S, b3.reshape(1, F))
  return out[:N]


# speedup vs baseline: 1.7827x; 1.7827x over previous
import functools

import jax
import jax.numpy as jnp
from jax import lax
from jax.experimental import pallas as pl
from jax.experimental.pallas import tpu as pltpu
from jax.experimental.pallas import tpu_sc as plsc

N = 10000
F = 128
H = 8
HH = 4
FH = HH * 16
C = 16
NEG = 0.2
NC = 2
NS = 16
NP = N + 112
K = 128
ZR = NP // NS

f32 = jnp.float32

_mesh = plsc.VectorSubcoreMesh(
    core_axis_name="c", subcore_axis_name="s", num_cores=NC, num_subcores=NS)


def _edge_pass(xl2, xr2, src, dst, att, nchunk):
  epw = nchunk * K

  @functools.partial(
      pl.kernel,
      out_type=(
          jax.ShapeDtypeStruct((NC, NP, FH), f32),
          jax.ShapeDtypeStruct((NC, NP, C), f32),
      ),
      mesh=_mesh,
      compiler_params=pltpu.CompilerParams(use_tc_tiling_on_sc=False),
      scratch_types=[
          pltpu.VMEM((2, K), jnp.int32),
          pltpu.VMEM((2, K), jnp.int32),
          pltpu.VMEM((2, K), jnp.int32),
          pltpu.VMEM((2, K), jnp.int32),
          pltpu.VMEM((2, K, FH), f32),
          pltpu.VMEM((2, K, FH), f32),
          pltpu.VMEM((2, K, FH), f32),
          pltpu.VMEM((2, K, C), f32),
          pltpu.VMEM((H, C), f32),
          pltpu.VMEM_SHARED((NP, FH), f32),
          pltpu.VMEM_SHARED((NP, C), f32),
          pltpu.SemaphoreType.DMA,
          pltpu.SemaphoreType.DMA,
          pltpu.SemaphoreType.DMA,
          pltpu.SemaphoreType.DMA,
          pltpu.SemaphoreType.DMA,
          pltpu.SemaphoreType.DMA,
          pltpu.SemaphoreType.DMA,
          pltpu.SemaphoreType.DMA,
      ],
  )
  def body(xl_h, xr_h, src_h, dst_h, att_h, wacc_h, den_h,
           srcv, dstv, dstv2, dsc, xlv, xrv, wv, eev, attv,
           wacc_sh, den_sh, gl0, gl1, gr0, gr1, sw0, sw1, se0, se1):
    gl = [gl0, gl1]
    gr = [gr0, gr1]
    sw = [sw0, sw1]
    se = [se0, se1]
    cid = lax.axis_index("c")
    sid = lax.axis_index("s")

    zvec = jnp.zeros((C,), f32)

    def zrow(e, carry):
      for j in range(FH // C):
        wv[0, e, pl.ds(j * C, C)] = zvec
      eev[0, e, :] = zvec
      return carry

    lax.fori_loop(0, K, zrow, None)

    zb = sid * ZR
    nfull = ZR // K
    rem = ZR - nfull * K
    for i in range(nfull):
      pltpu.sync_copy(wv.at[0], wacc_sh.at[pl.ds(zb + i * K, K)])
      pltpu.sync_copy(eev.at[0], den_sh.at[pl.ds(zb + i * K, K)])
    if rem:
      pltpu.sync_copy(wv.at[0, pl.ds(0, rem)],
                      wacc_sh.at[pl.ds(zb + nfull * K, rem)])
      pltpu.sync_copy(eev.at[0, pl.ds(0, rem)],
                      den_sh.at[pl.ds(zb + nfull * K, rem)])
    pltpu.sync_copy(att_h, attv)
    plsc.subcore_barrier()

    lane = lax.iota(jnp.int32, C)
    perm8, perm4, perm2, perm1 = (lane ^ 8, lane ^ 4, lane ^ 2, lane ^ 1)
    mask8 = lane < 8
    mask4 = (lane & 4) == 0
    bidx = [jnp.full((C,), (h % 2) * 8 + (h // 2) * 4, jnp.int32)
            for h in range(HH)]

    def bfly(v, p):
      return v + jnp.take_along_axis(v, p, axis=0)

    shift = jnp.full((C,), cid * NP, jnp.int32)
    hbase = cid * HH
    att_regs = [attv[hbase + h, :] for h in range(HH)]

    def prep(bi, k):
      eb = sid * epw + k * K
      pltpu.sync_copy(src_h.at[pl.ds(eb, K)], srcv.at[bi])
      pltpu.sync_copy(dst_h.at[pl.ds(eb, K)], dstv.at[bi])

      def shift_body(j, carry2):
        srcv[bi, pl.ds(j * C, C)] = srcv[bi, pl.ds(j * C, C)] + shift
        dstv2[bi, pl.ds(j * C, C)] = dstv[bi, pl.ds(j * C, C)] + shift
        return carry2

      lax.fori_loop(0, K // C, shift_body, None)
      pltpu.async_copy(xl_h.at[srcv.at[bi]], xlv.at[bi], gl[bi])
      pltpu.async_copy(xr_h.at[dstv2.at[bi]], xrv.at[bi], gr[bi])

    def process(bi, first):
      pltpu.make_async_copy(xl_h.at[srcv.at[bi]], xlv.at[bi], gl[bi]).wait()
      pltpu.make_async_copy(xr_h.at[dstv2.at[bi]], xrv.at[bi], gr[bi]).wait()

      @pl.when(jnp.logical_not(first))
      def _():
        pltpu.make_async_copy(wv.at[bi], wacc_sh.at[dsc.at[bi]], sw[bi]).wait()
        pltpu.make_async_copy(eev.at[bi], den_sh.at[dsc.at[bi]], se[bi]).wait()

      def snap(j, carry2):
        dsc[bi, pl.ds(j * C, C)] = dstv[bi, pl.ds(j * C, C)]
        return carry2

      lax.fori_loop(0, K // C, snap, None)

      @plsc.parallel_loop(0, K, unroll=4)
      def edge_body(e):
        a = [xlv[bi, e, pl.ds(h * C, C)] for h in range(HH)]
        p = []
        for h in range(HH):
          b = xrv[bi, e, pl.ds(h * C, C)]
          s = a[h] + b
          m = jnp.maximum(s, NEG * s)
          p.append(m * att_regs[h])
        v = [bfly(x, perm8) for x in p]
        m01 = jnp.where(mask8, v[0], v[1])
        m23 = jnp.where(mask8, v[2], v[3])
        q = jnp.where(mask4, bfly(m01, perm4), bfly(m23, perm4))
        ev4 = jnp.exp(bfly(bfly(q, perm2), perm1))
        eev[bi, e, :] = ev4
        for h in range(HH):
          wv[bi, e, pl.ds(h * C, C)] = (
              a[h] * jnp.take_along_axis(ev4, bidx[h], axis=0))

      pltpu.async_copy(wv.at[bi], wacc_sh.at[dsc.at[bi]], sw[bi], add=True)
      pltpu.async_copy(eev.at[bi], den_sh.at[dsc.at[bi]], se[bi], add=True)

    prep(0, 0)

    def pair_body(i, carry):
      k = 2 * i
      prep(1, k + 1)
      process(0, i == 0)

      @pl.when(k + 2 < nchunk)
      def _():
        prep(0, k + 2)

      process(1, i == 0)
      return carry

    lax.fori_loop(0, nchunk // 2, pair_body, None)
    for bi in range(2):
      pltpu.make_async_copy(wv.at[bi], wacc_sh.at[dsc.at[bi]], sw[bi]).wait()
      pltpu.make_async_copy(eev.at[bi], den_sh.at[dsc.at[bi]], se[bi]).wait()
    plsc.subcore_barrier()
    pltpu.sync_copy(wacc_sh.at[pl.ds(zb, ZR)], wacc_h.at[cid, pl.ds(zb, ZR)])
    pltpu.sync_copy(den_sh.at[pl.ds(zb, ZR)], den_h.at[cid, pl.ds(zb, ZR)])

  return body(xl2, xr2, src, dst, att)


def _mm2(xp, Wl, Wr):
  def bodyfn(x_ref, wl_ref, wr_ref, xl_ref, xr_ref):
    xv = x_ref[...]
    xl = jnp.dot(xv, wl_ref[...], preferred_element_type=f32)
    xr = jnp.dot(xv, wr_ref[...], preferred_element_type=f32)
    xl_ref[...] = jnp.concatenate([xl[:, :FH], xl[:, FH:]], axis=0)
    xr_ref[...] = jnp.concatenate([xr[:, :FH], xr[:, FH:]], axis=0)

  return pl.pallas_call(
      bodyfn,
      out_shape=(jax.ShapeDtypeStruct((NC * NP, FH), f32),
                 jax.ShapeDtypeStruct((NC * NP, FH), f32)),
  )(xp, Wl, Wr)


def _combine(wa_ref, dn_ref, s_ref, b_ref):
  wa = jnp.concatenate([wa_ref[0], wa_ref[1]], axis=1)
  rows = lax.broadcasted_iota(jnp.int32, (C, HH), 0)
  cols = lax.broadcasted_iota(jnp.int32, (C, HH), 1)
  sel = (rows == (cols % 2) * 8 + (cols // 2) * 4).astype(f32)
  d = jnp.concatenate(
      [jnp.dot(dn_ref[0], sel, preferred_element_type=f32),
       jnp.dot(dn_ref[1], sel, preferred_element_type=f32)], axis=1)
  r = 1.0 / (d + 1e-16)
  rexp = jnp.dot(r, s_ref[...], preferred_element_type=f32)
  return wa * rexp + b_ref[...]


def _combine_mm(wacc, den, S, b, Wl, Wr):
  def bodyfn(wa_ref, dn_ref, s_ref, b_ref, wl_ref, wr_ref, xl_ref, xr_ref):
    h = _combine(wa_ref, dn_ref, s_ref, b_ref)
    h = jnp.where(h > 0, h, jnp.exp(h) - 1.0)
    xl = jnp.dot(h, wl_ref[...], preferred_element_type=f32)
    xr = jnp.dot(h, wr_ref[...], preferred_element_type=f32)
    xl_ref[...] = jnp.concatenate([xl[:, :FH], xl[:, FH:]], axis=0)
    xr_ref[...] = jnp.concatenate([xr[:, :FH], xr[:, FH:]], axis=0)

  return pl.pallas_call(
      bodyfn,
      out_shape=(jax.ShapeDtypeStruct((NC * NP, FH), f32),
                 jax.ShapeDtypeStruct((NC * NP, FH), f32)),
  )(wacc, den, S, b, Wl, Wr)


def _finalize(wacc, den, S, b):
  def bodyfn(wa_ref, dn_ref, s_ref, b_ref, o_ref):
    h = _combine(wa_ref, dn_ref, s_ref, b_ref)
    m = jnp.max(h, axis=1, keepdims=True)
    z = h - m
    o_ref[...] = z - jnp.log(jnp.sum(jnp.exp(z), axis=1, keepdims=True))

  return pl.pallas_call(
      bodyfn,
      out_shape=jax.ShapeDtypeStruct((NP, F), f32),
  )(wacc, den, S, b)


def kernel(x, edge_index, W1l, W1r, att1, b1, W2l, W2r, att2, b2,
           W3l, W3r, att3, b3):
  eg = edge_index.shape[1]
  etot = eg + N
  nchunk = -(-etot // (NS * K))
  nchunk += nchunk & 1
  ep = NS * nchunk * K

  loop = jnp.arange(N, dtype=jnp.int32)
  padi = jnp.full((ep - etot,), N, jnp.int32)
  src = jnp.concatenate([edge_index[0].astype(jnp.int32), loop, padi])
  dst = jnp.concatenate([edge_index[1].astype(jnp.int32), loop, padi])

  xp = jnp.pad(x.astype(f32), ((0, NP - N), (0, 0)))

  S = (jnp.arange(F)[None, :] // C == jnp.arange(H)[:, None]).astype(f32)

  xl, xr = _mm2(xp, W1l, W1r)
  wacc, den = _edge_pass(xl, xr, src, dst, att1, nchunk)
  xl, xr = _combine_mm(wacc, den, S, b1.reshape(1, F), W2l, W2r)
  wacc, den = _edge_pass(xl, xr, src, dst, att2, nchunk)
  xl, xr = _combine_mm(wacc, den, S, b2.reshape(1, F), W3l, W3r)
  wacc, den = _edge_pass(xl, xr, src, dst, att3, nchunk)
  out = _finalize(wacc, den, S, b3.reshape(1, F))
  return out[:N]


# speedup vs baseline: 2.4744x; 1.3880x over previous
import functools

import jax
import jax.numpy as jnp
from jax import lax
from jax.experimental import pallas as pl
from jax.experimental.pallas import tpu as pltpu
from jax.experimental.pallas import tpu_sc as plsc

N = 10000
F = 128
H = 8
HH = 4
FH = HH * 16
C = 16
NEG = 0.2
NC = 2
NS = 16
NP = N + 112
K = 128
ZR = NP // NS

f32 = jnp.float32

_mesh = plsc.VectorSubcoreMesh(
    core_axis_name="c", subcore_axis_name="s", num_cores=NC, num_subcores=NS)


def _edge_pass(xl2, xr2, src, dst, att, nchunk):
  epw = nchunk * K

  @functools.partial(
      pl.kernel,
      out_type=(
          jax.ShapeDtypeStruct((NC, NP, FH), f32),
          jax.ShapeDtypeStruct((NC, NP, C), f32),
      ),
      mesh=_mesh,
      compiler_params=pltpu.CompilerParams(use_tc_tiling_on_sc=False),
      scratch_types=[
          pltpu.VMEM((2, K), jnp.int32),
          pltpu.VMEM((2, K), jnp.int32),
          pltpu.VMEM((2, K), jnp.int32),
          pltpu.VMEM((2, K), jnp.int32),
          pltpu.VMEM((2, K, FH), f32),
          pltpu.VMEM((2, K, FH), f32),
          pltpu.VMEM((2, K, FH), f32),
          pltpu.VMEM((2, K, C), f32),
          pltpu.VMEM((H, C), f32),
          pltpu.VMEM_SHARED((NP, FH), f32),
          pltpu.VMEM_SHARED((NP, C), f32),
          pltpu.SemaphoreType.DMA,
          pltpu.SemaphoreType.DMA,
          pltpu.SemaphoreType.DMA,
          pltpu.SemaphoreType.DMA,
          pltpu.SemaphoreType.DMA,
          pltpu.SemaphoreType.DMA,
          pltpu.SemaphoreType.DMA,
          pltpu.SemaphoreType.DMA,
          pltpu.SemaphoreType.DMA,
          pltpu.SemaphoreType.DMA,
      ],
  )
  def body(xl_h, xr_h, src_h, dst_h, att_h, wacc_h, den_h,
           srcv, dstv, dstv2, dsc, xlv, xrv, wv, eev, attv,
           wacc_sh, den_sh, gl0, gl1, gr0, gr1, sw0, sw1, se0, se1, gi0, gi1):
    gl = [gl0, gl1]
    gr = [gr0, gr1]
    sw = [sw0, sw1]
    se = [se0, se1]
    gi = [gi0, gi1]
    cid = lax.axis_index("c")
    sid = lax.axis_index("s")

    zvec = jnp.zeros((C,), f32)

    def zrow(e, carry):
      for j in range(FH // C):
        wv[0, e, pl.ds(j * C, C)] = zvec
      eev[0, e, :] = zvec
      return carry

    lax.fori_loop(0, K, zrow, None)

    zb = sid * ZR
    nfull = ZR // K
    rem = ZR - nfull * K
    for i in range(nfull):
      pltpu.sync_copy(wv.at[0], wacc_sh.at[pl.ds(zb + i * K, K)])
      pltpu.sync_copy(eev.at[0], den_sh.at[pl.ds(zb + i * K, K)])
    if rem:
      pltpu.sync_copy(wv.at[0, pl.ds(0, rem)],
                      wacc_sh.at[pl.ds(zb + nfull * K, rem)])
      pltpu.sync_copy(eev.at[0, pl.ds(0, rem)],
                      den_sh.at[pl.ds(zb + nfull * K, rem)])
    pltpu.sync_copy(att_h, attv)
    plsc.subcore_barrier()

    lane = lax.iota(jnp.int32, C)
    perm8, perm4, perm2, perm1 = (lane ^ 8, lane ^ 4, lane ^ 2, lane ^ 1)
    mask8 = lane < 8
    mask4 = (lane & 4) == 0
    bidx = [jnp.full((C,), (h % 2) * 8 + (h // 2) * 4, jnp.int32)
            for h in range(HH)]

    def bfly(v, p):
      return v + jnp.take_along_axis(v, p, axis=0)

    shift = jnp.full((C,), cid * NP, jnp.int32)
    hbase = cid * HH
    att_regs = [attv[hbase + h, :] for h in range(HH)]

    def prep_idx(bi, k):
      eb = sid * epw + k * K
      pltpu.async_copy(src_h.at[pl.ds(eb, K)], srcv.at[bi], gi[bi])
      pltpu.async_copy(dst_h.at[pl.ds(eb, K)], dstv.at[bi], gi[bi])

    def prep_go(bi, k):
      eb = sid * epw + k * K
      pltpu.make_async_copy(src_h.at[pl.ds(eb, K)], srcv.at[bi], gi[bi]).wait()
      pltpu.make_async_copy(dst_h.at[pl.ds(eb, K)], dstv.at[bi], gi[bi]).wait()

      def shift_body(j, carry2):
        srcv[bi, pl.ds(j * C, C)] = srcv[bi, pl.ds(j * C, C)] + shift
        dstv2[bi, pl.ds(j * C, C)] = dstv[bi, pl.ds(j * C, C)] + shift
        return carry2

      lax.fori_loop(0, K // C, shift_body, None)
      pltpu.async_copy(xl_h.at[srcv.at[bi]], xlv.at[bi], gl[bi])
      pltpu.async_copy(xr_h.at[dstv2.at[bi]], xrv.at[bi], gr[bi])

    def process(bi, first, next_k):
      pltpu.make_async_copy(xl_h.at[srcv.at[bi]], xlv.at[bi], gl[bi]).wait()
      pltpu.make_async_copy(xr_h.at[dstv2.at[bi]], xrv.at[bi], gr[bi]).wait()

      @pl.when(jnp.logical_not(first))
      def _():
        pltpu.make_async_copy(wv.at[bi], wacc_sh.at[dsc.at[bi]], sw[bi]).wait()
        pltpu.make_async_copy(eev.at[bi], den_sh.at[dsc.at[bi]], se[bi]).wait()

      def snap(j, carry2):
        dsc[bi, pl.ds(j * C, C)] = dstv[bi, pl.ds(j * C, C)]
        return carry2

      lax.fori_loop(0, K // C, snap, None)

      @pl.when(next_k < nchunk)
      def _():
        prep_idx(bi, next_k)

      @plsc.parallel_loop(0, K, unroll=4)
      def edge_body(e):
        a = [xlv[bi, e, pl.ds(h * C, C)] for h in range(HH)]
        p = []
        for h in range(HH):
          b = xrv[bi, e, pl.ds(h * C, C)]
          s = a[h] + b
          m = jnp.maximum(s, NEG * s)
          p.append(m * att_regs[h])
        v = [bfly(x, perm8) for x in p]
        m01 = jnp.where(mask8, v[0], v[1])
        m23 = jnp.where(mask8, v[2], v[3])
        q = jnp.where(mask4, bfly(m01, perm4), bfly(m23, perm4))
        ev4 = jnp.exp(bfly(bfly(q, perm2), perm1))
        eev[bi, e, :] = ev4
        for h in range(HH):
          wv[bi, e, pl.ds(h * C, C)] = (
              a[h] * jnp.take_along_axis(ev4, bidx[h], axis=0))

      pltpu.async_copy(wv.at[bi], wacc_sh.at[dsc.at[bi]], sw[bi], add=True)
      pltpu.async_copy(eev.at[bi], den_sh.at[dsc.at[bi]], se[bi], add=True)

    prep_idx(0, 0)
    prep_go(0, 0)
    prep_idx(1, 1)

    def pair_body(i, carry):
      k = 2 * i
      prep_go(1, k + 1)
      process(0, i == 0, k + 2)

      @pl.when(k + 2 < nchunk)
      def _():
        prep_go(0, k + 2)

      process(1, i == 0, k + 3)
      return carry

    lax.fori_loop(0, nchunk // 2, pair_body, None)
    for bi in range(2):
      pltpu.make_async_copy(wv.at[bi], wacc_sh.at[dsc.at[bi]], sw[bi]).wait()
      pltpu.make_async_copy(eev.at[bi], den_sh.at[dsc.at[bi]], se[bi]).wait()
    plsc.subcore_barrier()
    pltpu.sync_copy(wacc_sh.at[pl.ds(zb, ZR)], wacc_h.at[cid, pl.ds(zb, ZR)])
    pltpu.sync_copy(den_sh.at[pl.ds(zb, ZR)], den_h.at[cid, pl.ds(zb, ZR)])

  return body(xl2, xr2, src, dst, att)


def _mm2(xp, Wl, Wr):
  def bodyfn(x_ref, wl_ref, wr_ref, xl_ref, xr_ref):
    xv = x_ref[...]
    xl = jnp.dot(xv, wl_ref[...], preferred_element_type=f32)
    xr = jnp.dot(xv, wr_ref[...], preferred_element_type=f32)
    xl_ref[...] = jnp.concatenate([xl[:, :FH], xl[:, FH:]], axis=0)
    xr_ref[...] = jnp.concatenate([xr[:, :FH], xr[:, FH:]], axis=0)

  return pl.pallas_call(
      bodyfn,
      out_shape=(jax.ShapeDtypeStruct((NC * NP, FH), f32),
                 jax.ShapeDtypeStruct((NC * NP, FH), f32)),
  )(xp, Wl, Wr)


def _combine(wa_ref, dn_ref, s_ref, b_ref):
  wa = jnp.concatenate([wa_ref[0], wa_ref[1]], axis=1)
  rows = lax.broadcasted_iota(jnp.int32, (C, HH), 0)
  cols = lax.broadcasted_iota(jnp.int32, (C, HH), 1)
  sel = (rows == (cols % 2) * 8 + (cols // 2) * 4).astype(f32)
  d = jnp.concatenate(
      [jnp.dot(dn_ref[0], sel, preferred_element_type=f32),
       jnp.dot(dn_ref[1], sel, preferred_element_type=f32)], axis=1)
  r = 1.0 / (d + 1e-16)
  rexp = jnp.dot(r, s_ref[...], preferred_element_type=f32)
  return wa * rexp + b_ref[...]


def _combine_mm(wacc, den, S, b, Wl, Wr):
  def bodyfn(wa_ref, dn_ref, s_ref, b_ref, wl_ref, wr_ref, xl_ref, xr_ref):
    h = _combine(wa_ref, dn_ref, s_ref, b_ref)
    h = jnp.where(h > 0, h, jnp.exp(h) - 1.0)
    xl = jnp.dot(h, wl_ref[...], preferred_element_type=f32)
    xr = jnp.dot(h, wr_ref[...], preferred_element_type=f32)
    xl_ref[...] = jnp.concatenate([xl[:, :FH], xl[:, FH:]], axis=0)
    xr_ref[...] = jnp.concatenate([xr[:, :FH], xr[:, FH:]], axis=0)

  return pl.pallas_call(
      bodyfn,
      out_shape=(jax.ShapeDtypeStruct((NC * NP, FH), f32),
                 jax.ShapeDtypeStruct((NC * NP, FH), f32)),
  )(wacc, den, S, b, Wl, Wr)


def _finalize(wacc, den, S, b):
  def bodyfn(wa_ref, dn_ref, s_ref, b_ref, o_ref):
    h = _combine(wa_ref, dn_ref, s_ref, b_ref)
    m = jnp.max(h, axis=1, keepdims=True)
    z = h - m
    o_ref[...] = z - jnp.log(jnp.sum(jnp.exp(z), axis=1, keepdims=True))

  return pl.pallas_call(
      bodyfn,
      out_shape=jax.ShapeDtypeStruct((NP, F), f32),
  )(wacc, den, S, b)


def kernel(x, edge_index, W1l, W1r, att1, b1, W2l, W2r, att2, b2,
           W3l, W3r, att3, b3):
  eg = edge_index.shape[1]
  etot = eg + N
  nchunk = -(-etot // (NS * K))
  nchunk += nchunk & 1
  ep = NS * nchunk * K

  loop = jnp.arange(N, dtype=jnp.int32)
  padi = jnp.full((ep - etot,), N, jnp.int32)
  src = jnp.concatenate([edge_index[0].astype(jnp.int32), loop, padi])
  dst = jnp.concatenate([edge_index[1].astype(jnp.int32), loop, padi])

  xp = jnp.pad(x.astype(f32), ((0, NP - N), (0, 0)))

  S = (jnp.arange(F)[None, :] // C == jnp.arange(H)[:, None]).astype(f32)

  xl, xr = _mm2(xp, W1l, W1r)
  wacc, den = _edge_pass(xl, xr, src, dst, att1, nchunk)
  xl, xr = _combine_mm(wacc, den, S, b1.reshape(1, F), W2l, W2r)
  wacc, den = _edge_pass(xl, xr, src, dst, att2, nchunk)
  xl, xr = _combine_mm(wacc, den, S, b2.reshape(1, F), W3l, W3r)
  wacc, den = _edge_pass(xl, xr, src, dst, att3, nchunk)
  out = _finalize(wacc, den, S, b3.reshape(1, F))
  return out[:N]
